# baseline (speedup 1.0000x reference)
import functools

import jax
import jax.numpy as jnp
from jax import lax
from jax.experimental import pallas as pl
from jax.experimental.pallas import tpu as pltpu
from jax.experimental.pallas import tpu_sc as plsc

_CHUNK = 312


def kernel(embed_user, embed_item):
    n, d = embed_user.shape
    info = plsc.get_sparse_core_info()
    nw = info.num_cores * info.num_subcores
    rows = (n // nw) // 8 * 8
    rem = n - nw * rows
    chunk = _CHUNK
    nchunks = rows // chunk
    assert rows % chunk == 0

    mesh = plsc.VectorSubcoreMesh(core_axis_name="c", subcore_axis_name="s")

    @functools.partial(
        pl.kernel,
        mesh=mesh,
        out_type=jax.ShapeDtypeStruct((2, n, d), embed_user.dtype),
        scratch_types=[
            pltpu.VMEM((2, chunk, d), jnp.float32),
            pltpu.SemaphoreType.DMA,
            pltpu.SemaphoreType.DMA,
            pltpu.SemaphoreType.DMA,
            pltpu.SemaphoreType.DMA,
        ],
    )
    def copy_tables(user_hbm, item_hbm, out_hbm, buf,
                    sem_in0, sem_in1, sem_out0, sem_out1):
        wid = lax.axis_index("s") * info.num_cores + lax.axis_index("c")
        base = wid * rows
        srcs = (user_hbm, item_hbm)
        sem_in = (sem_in0, sem_in1)
        sem_out = (sem_out0, sem_out1)

        def src_dst(k, tbase, tchunk):
            t, c = divmod(k, nchunks)
            lo = tbase + c * chunk
            return (srcs[t].at[pl.ds(lo, tchunk)],
                    out_hbm.at[t, pl.ds(lo, tchunk)])

        total = 2 * nchunks
        loads = [None] * total
        stores = [None] * total
        for k in range(total):
            s, o = src_dst(k, base, chunk)
            loads[k] = pltpu.make_async_copy(s, buf.at[k % 2], sem_in[k % 2])
            stores[k] = pltpu.make_async_copy(buf.at[k % 2], o, sem_out[k % 2])

        loads[0].start()
        for k in range(total):
            if k + 1 < total:
                if k >= 1:
                    stores[k - 1].wait()
                loads[k + 1].start()
            loads[k].wait()
            stores[k].start()
        stores[total - 2].wait()
        stores[total - 1].wait()

        if rem:
            tail = nw * rows

            @pl.when(wid < 2)
            def _():
                for t in range(2):
                    @pl.when(wid == t)
                    def _():
                        lt = pltpu.make_async_copy(
                            srcs[t].at[pl.ds(tail, rem)],
                            buf.at[0, pl.ds(0, rem)], sem_in[0])
                        st = pltpu.make_async_copy(
                            buf.at[0, pl.ds(0, rem)],
                            out_hbm.at[t, pl.ds(tail, rem)], sem_out[0])
                        lt.start()
                        lt.wait()
                        st.start()
                        st.wait()

    return copy_tables(embed_user, embed_item)


# speedup vs baseline: 1.0065x; 1.0065x over previous
import functools

import jax
import jax.numpy as jnp
from jax import lax
from jax.experimental import pallas as pl
from jax.experimental.pallas import tpu as pltpu
from jax.experimental.pallas import tpu_sc as plsc

_CHUNK = 208
_DEPTH = 3


def kernel(embed_user, embed_item):
    n, d = embed_user.shape
    info = plsc.get_sparse_core_info()
    nw = info.num_cores * info.num_subcores
    rows = (n // nw) // 8 * 8
    rem = n - nw * rows
    chunk = _CHUNK
    depth = _DEPTH
    nchunks = rows // chunk
    assert rows % chunk == 0

    mesh = plsc.VectorSubcoreMesh(core_axis_name="c", subcore_axis_name="s")

    @functools.partial(
        pl.kernel,
        mesh=mesh,
        out_type=jax.ShapeDtypeStruct((2, n, d), embed_user.dtype),
        scratch_types=(
            [pltpu.VMEM((depth, chunk, d), jnp.float32)]
            + [pltpu.SemaphoreType.DMA] * (2 * depth)
        ),
    )
    def copy_tables(user_hbm, item_hbm, out_hbm, buf, *sems):
        sem_in, sem_out = sems[:depth], sems[depth:]
        wid = lax.axis_index("s") * info.num_cores + lax.axis_index("c")
        base = wid * rows
        srcs = (user_hbm, item_hbm)

        def src_dst(k):
            t, c = divmod(k, nchunks)
            lo = base + c * chunk
            return (srcs[t].at[pl.ds(lo, chunk)],
                    out_hbm.at[t, pl.ds(lo, chunk)])

        total = 2 * nchunks
        loads = [None] * total
        stores = [None] * total
        for k in range(total):
            s, o = src_dst(k)
            p = k % depth
            loads[k] = pltpu.make_async_copy(s, buf.at[p], sem_in[p])
            stores[k] = pltpu.make_async_copy(buf.at[p], o, sem_out[p])

        for k in range(depth):
            loads[k].start()
        for k in range(total):
            if k >= 1 and k - 1 + depth < total:
                stores[k - 1].wait()
                loads[k - 1 + depth].start()
            loads[k].wait()
            stores[k].start()
        for k in range(max(0, total - depth), total):
            stores[k].wait()

        if rem:
            tail = nw * rows

            for t in range(2):
                @pl.when(wid == t)
                def _():
                    lt = pltpu.make_async_copy(
                        srcs[t].at[pl.ds(tail, rem)],
                        buf.at[0, pl.ds(0, rem)], sem_in[0])
                    st = pltpu.make_async_copy(
                        buf.at[0, pl.ds(0, rem)],
                        out_hbm.at[t, pl.ds(tail, rem)], sem_out[0])
                    lt.start()
                    lt.wait()
                    st.start()
                    st.wait()

    return copy_tables(embed_user, embed_item)


# speedup vs baseline: 1.4499x; 1.4406x over previous
import jax
import jax.numpy as jnp
from jax.experimental import pallas as pl
from jax.experimental.pallas import tpu as pltpu

_CHUNK_ROWS = 4000
_DEPTH = 6
_SLACK = 2


def kernel(embed_user, embed_item):
    n, d = embed_user.shape
    chunk = _CHUNK_ROWS if n % _CHUNK_ROWS == 0 else n
    nchunks = n // chunk
    total = 2 * nchunks
    depth = min(_DEPTH, total)
    slack = min(_SLACK, depth - 1)

    def body(user_hbm, item_hbm, out_hbm, buf, *sems):
        sem_in, sem_out = sems[:depth], sems[depth:]
        srcs = (user_hbm, item_hbm)

        def mk(k):
            t, c = divmod(k, nchunks)
            p = k % depth
            lo = c * chunk
            load = pltpu.make_async_copy(
                srcs[t].at[pl.ds(lo, chunk)], buf.at[p], sem_in[p])
            store = pltpu.make_async_copy(
                buf.at[p], out_hbm.at[t, pl.ds(lo, chunk)], sem_out[p])
            return load, store

        ops = [mk(k) for k in range(total)]
        for k in range(depth):
            ops[k][0].start()
        for k in range(total):
            if k >= slack and k - slack + depth < total:
                ops[k - slack][1].wait()
                ops[k - slack + depth][0].start()
            ops[k][0].wait()
            ops[k][1].start()
        for k in range(max(0, total - depth), total):
            ops[k][1].wait()

    return pl.pallas_call(
        body,
        out_shape=jax.ShapeDtypeStruct((2, n, d), embed_user.dtype),
        in_specs=[
            pl.BlockSpec(memory_space=pltpu.MemorySpace.HBM),
            pl.BlockSpec(memory_space=pltpu.MemorySpace.HBM),
        ],
        out_specs=pl.BlockSpec(memory_space=pltpu.MemorySpace.HBM),
        scratch_shapes=(
            [pltpu.VMEM((depth, chunk, d), embed_user.dtype)]
            + [pltpu.SemaphoreType.DMA] * (2 * depth)
        ),
    )(embed_user, embed_item)
